# baseline (speedup 1.0000x reference)
import functools

import jax
import jax.numpy as jnp
import numpy as np
from jax import lax
from jax.experimental import pallas as pl
from jax.experimental.pallas import tpu as pltpu
from jax.experimental.pallas import tpu_sc as plsc

_INTERPRET = False

_SC_CHUNK = 128
_SC_WORKERS = 32

_NEG = -3.0e38


def _prep_body(x_ref, g_ref, b_ref, wqt_ref, wkt_ref, wvt_ref, poolt_ref,
               xn_ref, qkvt_ref, centt_ref):
    xb = x_ref[0]
    mu = jnp.mean(xb, axis=0, keepdims=True)
    var = jnp.mean((xb - mu) * (xb - mu), axis=0, keepdims=True)
    xn = (xb - mu) / jnp.sqrt(var + 1e-6) * g_ref[:] + b_ref[:]
    xn_ref[0] = xn
    xnt = xn.T
    qt = jnp.dot(xnt, wqt_ref[:], preferred_element_type=jnp.float32)
    kt = jnp.dot(xnt, wkt_ref[:], preferred_element_type=jnp.float32)
    vt = jnp.dot(xnt, wvt_ref[:], preferred_element_type=jnp.float32)
    qkvt_ref[0] = jnp.concatenate([qt, kt, vt], axis=1)
    centt_ref[0] = jnp.dot(poolt_ref[:], xnt, preferred_element_type=jnp.float32,
                           precision=lax.Precision.HIGHEST)


def _prep(x, gamma, beta, Wq, Wk, Wv, pool_t, B, C, N, Nt, Kc):
    grid = (B, N // Nt)
    kcells = pool_t.shape[0]
    return pl.pallas_call(
        _prep_body,
        grid=grid,
        in_specs=[
            pl.BlockSpec((1, C, Nt), lambda b, t: (b, 0, t)),
            pl.BlockSpec((C, 1), lambda b, t: (0, 0)),
            pl.BlockSpec((C, 1), lambda b, t: (0, 0)),
            pl.BlockSpec((C, C), lambda b, t: (0, 0)),
            pl.BlockSpec((C, C), lambda b, t: (0, 0)),
            pl.BlockSpec((C, C), lambda b, t: (0, 0)),
            pl.BlockSpec((kcells, Nt), lambda b, t: (0, 0)),
        ],
        out_specs=[
            pl.BlockSpec((1, C, Nt), lambda b, t: (b, 0, t)),
            pl.BlockSpec((1, Nt, 3 * C), lambda b, t: (b, t, 0)),
            pl.BlockSpec((1, kcells, C), lambda b, t: (b, t, 0)),
        ],
        out_shape=[
            jax.ShapeDtypeStruct((B, C, N), jnp.float32),
            jax.ShapeDtypeStruct((B, N, 3 * C), jnp.float32),
            jax.ShapeDtypeStruct((B, Kc, C), jnp.float32),
        ],
        interpret=_INTERPRET,
    )(x, gamma, beta, Wq, Wk, Wv, pool_t)


def _topk_body(centt_ref, xn_ref, sims_ref, inds_ref, w_ref, *, n, topk, rows):
    w_ref[:] = jnp.dot(centt_ref[0], xn_ref[0],
                       preferred_element_type=jnp.float32)

    iota = lax.broadcasted_iota(jnp.int32, (rows, n), 1)
    for i in range(topk):
        w = w_ref[:]
        m = jnp.max(w, axis=1, keepdims=True)
        idx = jnp.min(jnp.where(w == m, iota, n), axis=1, keepdims=True)
        w_ref[:] = jnp.where(iota == idx, _NEG, w)
        sims_ref[0, :, i:i + 1] = m
        inds_ref[0, :, i:i + 1] = idx


def _topk(centt, xn, B, C, Kc, N, topk, rows):
    grid = (B * (Kc // rows),)
    nb = Kc // rows
    body = functools.partial(_topk_body, n=N, topk=topk, rows=rows)
    return pl.pallas_call(
        body,
        grid=grid,
        in_specs=[
            pl.BlockSpec((1, rows, C), lambda i: (i // nb, i % nb, 0)),
            pl.BlockSpec((1, C, N), lambda i: (i // nb, 0, 0)),
        ],
        out_specs=[
            pl.BlockSpec((1, rows, topk), lambda i: (i // nb, i % nb, 0)),
            pl.BlockSpec((1, rows, topk), lambda i: (i // nb, i % nb, 0)),
        ],
        out_shape=[
            jax.ShapeDtypeStruct((B, Kc, topk), jnp.float32),
            jax.ShapeDtypeStruct((B, Kc, topk), jnp.int32),
        ],
        scratch_shapes=[pltpu.VMEM((rows, N), jnp.float32)],
        interpret=_INTERPRET,
    )(centt, xn)


def _attn_body(g_ref, jp_ref, sims_ref, o_ref, jpd_ref, sflat_ref, *, C, topk,
               gk, heads, scale):
    R = gk * topk
    hd = C // heads
    g = g_ref[0]
    sims = sims_ref[0]
    sm = jnp.max(sims, axis=1, keepdims=True)
    se = jnp.exp(sims - sm)
    sims_n = se / jnp.sum(se, axis=1, keepdims=True)
    for s in range(gk):
        sflat_ref[0:1, s * topk:(s + 1) * topk] = sims_n[s:s + 1, :]
    sims_flat = sflat_ref[:]

    jpd_ref[:] = jnp.zeros((R, R), jnp.float32)
    for s in range(gk):
        jpd_ref[s * topk:(s + 1) * topk,
                s * topk:(s + 1) * topk] = jp_ref[0, s] * scale
    jpd = jpd_ref[:]

    bi = lax.broadcasted_iota(jnp.int32, (R, R), 0) // topk
    bj = lax.broadcasted_iota(jnp.int32, (R, R), 1) // topk
    mask = bi == bj

    q_all = g[:, 0:C]
    k_all = g[:, C:2 * C]
    v_all = g[:, 2 * C:3 * C]

    for h in range(heads):
        qh = q_all[:, h * hd:(h + 1) * hd]
        kh = k_all[:, h * hd:(h + 1) * hd]
        vh = v_all[:, h * hd:(h + 1) * hd]
        s_mat = lax.dot_general(qh, kh, (((1,), (1,)), ((), ())),
                                preferred_element_type=jnp.float32)
        s_mat = s_mat * jpd
        s_mat = jnp.where(mask, s_mat, _NEG)
        p = jnp.exp(s_mat - jnp.max(s_mat, axis=1, keepdims=True))
        a = p / jnp.sum(p, axis=1, keepdims=True)
        a = a * sims_flat
        out_h = jnp.dot(a, vh, preferred_element_type=jnp.float32)
        wgt_h = jnp.sum(a, axis=1, keepdims=True)
        o_ref[0, :, h * hd:(h + 1) * hd] = out_h
        o_ref[0, :, C + h:C + h + 1] = wgt_h
    o_ref[0, :, C + heads:] = jnp.zeros((R, 16 - heads), jnp.float32)


def _attention(gath, jp, sims, B, C, Kc, topk, heads, gk):
    ow = C + 16
    scale = (C // heads) ** -0.5
    R = gk * topk
    body = functools.partial(_attn_body, C=C, topk=topk, gk=gk, heads=heads,
                             scale=scale)
    grid = (B, Kc // gk)
    out = pl.pallas_call(
        body,
        grid=grid,
        in_specs=[
            pl.BlockSpec((1, R, 3 * C), lambda b, t: (b, t, 0)),
            pl.BlockSpec((1, gk, topk, topk), lambda b, t: (b, t, 0, 0)),
            pl.BlockSpec((1, gk, topk), lambda b, t: (b, t, 0)),
        ],
        out_specs=pl.BlockSpec((1, R, ow), lambda b, t: (b, t, 0)),
        out_shape=jax.ShapeDtypeStruct((B, Kc * topk, ow), jnp.float32),
        scratch_shapes=[pltpu.VMEM((R, R), jnp.float32),
                        pltpu.VMEM((1, R), jnp.float32)],
        interpret=_INTERPRET,
    )(gath.reshape(B, Kc * topk, 3 * C), jp, sims)
    return out


def _combine_body(nd_ref, vt_ref, res_ref, *, C, heads):
    hd = C // heads
    nd = nd_ref[0]
    num = nd[:, 0:C]
    vt = vt_ref[0]
    den = jnp.concatenate([
        jnp.broadcast_to(nd[:, C + h:C + h + 1], num[:, :hd].shape)
        for h in range(heads)], axis=1)
    res = jnp.where(den > 1e-6, num / jnp.maximum(den, 1e-6), vt)
    res_ref[0] = res.T


def _combine(numden, vt, B, C, N, heads, Nt):
    body = functools.partial(_combine_body, C=C, heads=heads)
    grid = (B, N // Nt)
    return pl.pallas_call(
        body,
        grid=grid,
        in_specs=[
            pl.BlockSpec((1, Nt, C + 16), lambda b, t: (b, t, 0)),
            pl.BlockSpec((1, Nt, C), lambda b, t: (b, t, 0)),
        ],
        out_specs=pl.BlockSpec((1, C, Nt), lambda b, t: (b, 0, t)),
        out_shape=jax.ShapeDtypeStruct((B, C, N), jnp.float32),
        interpret=_INTERPRET,
    )(numden, vt)


def _sc_gather(table, gidx, D):
    M = gidx.shape[0]
    per_w = M // _SC_WORKERS
    nch = per_w // _SC_CHUNK
    mesh = plsc.VectorSubcoreMesh(core_axis_name="c", subcore_axis_name="s")

    @functools.partial(
        pl.kernel, mesh=mesh,
        out_type=jax.ShapeDtypeStruct((M, D), jnp.float32),
        scratch_types=[pltpu.VMEM((_SC_CHUNK,), jnp.int32),
                       pltpu.VMEM((_SC_CHUNK, D), jnp.float32),
                       pltpu.SemaphoreType.DMA],
        compiler_params=pltpu.CompilerParams(use_tc_tiling_on_sc=False),
    )
    def k(table_hbm, gidx_hbm, out_hbm, idx_v, rows_v, sem):
        wid = lax.axis_index("s") * 2 + lax.axis_index("c")
        base = wid * per_w

        def chunk(i, carry):
            off = base + i * _SC_CHUNK
            pltpu.sync_copy(gidx_hbm.at[pl.ds(off, _SC_CHUNK)], idx_v)
            pltpu.async_copy(table_hbm.at[idx_v], rows_v, sem).wait()
            pltpu.sync_copy(rows_v, out_hbm.at[pl.ds(off, _SC_CHUNK)])
            return carry

        lax.fori_loop(0, nch, chunk, 0)

    return k(table, gidx)


def _sc_scatter(contrib, inds, zrow, N):
    B, R, OW = contrib.shape
    per_t = R // 16
    nch = per_t // _SC_CHUNK
    half = N // 2
    acc_per_t = half // 16
    nz = acc_per_t // _SC_CHUNK
    mesh = plsc.VectorSubcoreMesh(core_axis_name="c", subcore_axis_name="s")

    @functools.partial(
        pl.kernel, mesh=mesh,
        out_type=jax.ShapeDtypeStruct((B, N, OW), jnp.float32),
        scratch_types=[pltpu.VMEM((_SC_CHUNK,), jnp.int32),
                       pltpu.VMEM((_SC_CHUNK,), jnp.int32),
                       pltpu.VMEM((_SC_CHUNK, OW), jnp.float32),
                       pltpu.VMEM((_SC_CHUNK, OW), jnp.float32),
                       pltpu.VMEM_SHARED((half + _SC_CHUNK, OW), jnp.float32),
                       pltpu.SemaphoreType.DMA],
        compiler_params=pltpu.CompilerParams(use_tc_tiling_on_sc=False),
    )
    def k(contrib_hbm, inds_hbm, zrow_hbm, out_hbm, idx_v, idx2_v, rows_v,
          zbuf, acc, sem):
        cid = lax.axis_index("c")
        sid = lax.axis_index("s")
        pltpu.sync_copy(zrow_hbm, zbuf)
        for j in range(B // 2):
            b = cid * (B // 2) + j
            for lo in (0, half):

                def zero(i, carry):
                    pltpu.sync_copy(
                        zbuf, acc.at[pl.ds(sid * acc_per_t + i * _SC_CHUNK,
                                           _SC_CHUNK)])
                    return carry

                lax.fori_loop(0, nz, zero, 0)
                plsc.subcore_barrier()

                def chunk(i, carry):
                    off = sid * per_t + i * _SC_CHUNK
                    pltpu.sync_copy(inds_hbm.at[b].at[pl.ds(off, _SC_CHUNK)],
                                    idx_v)
                    pltpu.sync_copy(contrib_hbm.at[b].at[pl.ds(off,
                                                               _SC_CHUNK)],
                                    rows_v)
                    for t in range(_SC_CHUNK // 16):
                        iv = idx_v[pl.ds(t * 16, 16)]
                        inr = jnp.logical_and(iv >= lo, iv < lo + half)
                        adj = jnp.where(inr, iv - lo, half + (iv & 127))
                        idx2_v[pl.ds(t * 16, 16)] = adj
                    pltpu.sync_copy(rows_v, acc.at[idx2_v], add=True)
                    return carry

                lax.fori_loop(0, nch, chunk, 0)
                plsc.subcore_barrier()
                pltpu.sync_copy(
                    acc.at[pl.ds(sid * acc_per_t, acc_per_t)],
                    out_hbm.at[b].at[pl.ds(lo + sid * acc_per_t, acc_per_t)])
                plsc.subcore_barrier()

    return k(contrib, inds, zrow)


def kernel(x, joint_probs, norm_gamma, norm_beta, Wq, Wk, Wv):
    B, C, H, W = x.shape
    N = H * W
    Kc = joint_probs.shape[1]
    topk = joint_probs.shape[2]
    heads = 4
    gh = gw = int(np.sqrt(Kc))
    ph, pw = H // gh, W // gw

    Nt = 2048
    rows_per_tile = Nt // W
    cells_per_tile = (rows_per_tile // ph) * gw

    nloc = np.arange(Nt)
    hl, wl = nloc // W, nloc % W
    cell = (hl // ph) * gw + (wl // pw)
    pool_t = np.zeros((cells_per_tile, Nt), np.float32)
    pool_t[cell, nloc] = 1.0 / (ph * pw)
    pool_t = jnp.asarray(pool_t)

    xf = x.reshape(B, C, N)
    xn, qkvt, centt = _prep(
        xf, norm_gamma.reshape(C, 1), norm_beta.reshape(C, 1),
        Wq.T, Wk.T, Wv.T, pool_t, B, C, N, Nt, Kc)

    sims, inds = _topk(centt, xn, B, C, Kc, N, topk, rows=64)

    gidx = (inds.reshape(B, Kc * topk)
            + (jnp.arange(B, dtype=jnp.int32) * N)[:, None]).reshape(-1)
    gath = _sc_gather(qkvt.reshape(B * N, 3 * C), gidx, 3 * C)

    o = _attention(gath.reshape(B, Kc * topk, 3 * C), joint_probs, sims,
                   B, C, Kc, topk, heads, gk=8)

    zrow = jnp.zeros((_SC_CHUNK, C + 16), jnp.float32)
    numden = _sc_scatter(o, inds.reshape(B, Kc * topk), zrow, N)

    res = _combine(numden, qkvt[..., 2 * C:3 * C], B, C, N, heads, Nt=2048)
    return res.reshape(B, C, H, W)


# speedup vs baseline: 1.0800x; 1.0800x over previous
import functools

import jax
import jax.numpy as jnp
import numpy as np
from jax import lax
from jax.experimental import pallas as pl
from jax.experimental.pallas import tpu as pltpu
from jax.experimental.pallas import tpu_sc as plsc

_INTERPRET = False

_SC_CHUNK = 128
_SC_WORKERS = 32

_NEG = -3.0e38


def _prep_body(x_ref, g_ref, b_ref, wqt_ref, wkt_ref, wvt_ref, poolt_ref,
               xn_ref, qkvt_ref, centt_ref):
    xb = x_ref[0]
    mu = jnp.mean(xb, axis=0, keepdims=True)
    var = jnp.mean((xb - mu) * (xb - mu), axis=0, keepdims=True)
    xn = (xb - mu) / jnp.sqrt(var + 1e-6) * g_ref[:] + b_ref[:]
    xn_ref[0] = xn
    xnt = xn.T
    qt = jnp.dot(xnt, wqt_ref[:], preferred_element_type=jnp.float32)
    kt = jnp.dot(xnt, wkt_ref[:], preferred_element_type=jnp.float32)
    vt = jnp.dot(xnt, wvt_ref[:], preferred_element_type=jnp.float32)
    qkvt_ref[0] = jnp.concatenate([qt, kt, vt], axis=1)
    centt_ref[0] = jnp.dot(poolt_ref[:], xnt, preferred_element_type=jnp.float32,
                           precision=lax.Precision.HIGHEST)


def _prep(x, gamma, beta, Wq, Wk, Wv, pool_t, B, C, N, Nt, Kc):
    grid = (B, N // Nt)
    kcells = pool_t.shape[0]
    return pl.pallas_call(
        _prep_body,
        grid=grid,
        in_specs=[
            pl.BlockSpec((1, C, Nt), lambda b, t: (b, 0, t)),
            pl.BlockSpec((C, 1), lambda b, t: (0, 0)),
            pl.BlockSpec((C, 1), lambda b, t: (0, 0)),
            pl.BlockSpec((C, C), lambda b, t: (0, 0)),
            pl.BlockSpec((C, C), lambda b, t: (0, 0)),
            pl.BlockSpec((C, C), lambda b, t: (0, 0)),
            pl.BlockSpec((kcells, Nt), lambda b, t: (0, 0)),
        ],
        out_specs=[
            pl.BlockSpec((1, C, Nt), lambda b, t: (b, 0, t)),
            pl.BlockSpec((1, Nt, 3 * C), lambda b, t: (b, t, 0)),
            pl.BlockSpec((1, kcells, C), lambda b, t: (b, t, 0)),
        ],
        out_shape=[
            jax.ShapeDtypeStruct((B, C, N), jnp.float32),
            jax.ShapeDtypeStruct((B, N, 3 * C), jnp.float32),
            jax.ShapeDtypeStruct((B, Kc, C), jnp.float32),
        ],
        interpret=_INTERPRET,
    )(x, gamma, beta, Wq, Wk, Wv, pool_t)


def _topk_body(centt_ref, xn_ref, sims_ref, inds_ref, w_ref, *, n, topk, rows):
    w_ref[:] = jnp.dot(centt_ref[0], xn_ref[0],
                       preferred_element_type=jnp.float32)

    iota = lax.broadcasted_iota(jnp.int32, (rows, n), 1)
    for i in range(topk):
        w = w_ref[:]
        m = jnp.max(w, axis=1, keepdims=True)
        idx = jnp.min(jnp.where(w == m, iota, n), axis=1, keepdims=True)
        w_ref[:] = jnp.where(iota == idx, _NEG, w)
        sims_ref[0, :, i:i + 1] = m
        inds_ref[0, :, i:i + 1] = idx


def _topk(centt, xn, B, C, Kc, N, topk, rows):
    grid = (B * (Kc // rows),)
    nb = Kc // rows
    body = functools.partial(_topk_body, n=N, topk=topk, rows=rows)
    return pl.pallas_call(
        body,
        grid=grid,
        in_specs=[
            pl.BlockSpec((1, rows, C), lambda i: (i // nb, i % nb, 0)),
            pl.BlockSpec((1, C, N), lambda i: (i // nb, 0, 0)),
        ],
        out_specs=[
            pl.BlockSpec((1, rows, topk), lambda i: (i // nb, i % nb, 0)),
            pl.BlockSpec((1, rows, topk), lambda i: (i // nb, i % nb, 0)),
        ],
        out_shape=[
            jax.ShapeDtypeStruct((B, Kc, topk), jnp.float32),
            jax.ShapeDtypeStruct((B, Kc, topk), jnp.int32),
        ],
        scratch_shapes=[pltpu.VMEM((rows, N), jnp.float32)],
        interpret=_INTERPRET,
    )(centt, xn)


def _attn_body(g_ref, jp_ref, sims_ref, o_ref, jpd_ref, sflat_ref, *, C, topk,
               gk, heads, scale):
    R = gk * topk
    hd = C // heads
    g = g_ref[0]
    sims = sims_ref[0]
    sm = jnp.max(sims, axis=1, keepdims=True)
    se = jnp.exp(sims - sm)
    sims_n = se / jnp.sum(se, axis=1, keepdims=True)
    for s in range(gk):
        sflat_ref[0:1, s * topk:(s + 1) * topk] = sims_n[s:s + 1, :]
    sims_flat = sflat_ref[:]

    jpd_ref[:] = jnp.zeros((R, R), jnp.float32)
    for s in range(gk):
        jpd_ref[s * topk:(s + 1) * topk,
                s * topk:(s + 1) * topk] = jp_ref[0, s] * scale
    jpd = jpd_ref[:]

    bi = lax.broadcasted_iota(jnp.int32, (R, R), 0) // topk
    bj = lax.broadcasted_iota(jnp.int32, (R, R), 1) // topk
    mask = bi == bj

    q_all = g[:, 0:C]
    k_all = g[:, C:2 * C]
    v_all = g[:, 2 * C:3 * C]

    for h in range(heads):
        qh = q_all[:, h * hd:(h + 1) * hd]
        kh = k_all[:, h * hd:(h + 1) * hd]
        vh = v_all[:, h * hd:(h + 1) * hd]
        s_mat = lax.dot_general(qh, kh, (((1,), (1,)), ((), ())),
                                preferred_element_type=jnp.float32)
        s_mat = s_mat * jpd
        s_mat = jnp.where(mask, s_mat, _NEG)
        p = jnp.exp(s_mat - jnp.max(s_mat, axis=1, keepdims=True))
        a = p / jnp.sum(p, axis=1, keepdims=True)
        a = a * sims_flat
        out_h = jnp.dot(a, vh, preferred_element_type=jnp.float32)
        wgt_h = jnp.sum(a, axis=1, keepdims=True)
        o_ref[0, :, h * hd:(h + 1) * hd] = out_h
        o_ref[0, :, C + h:C + h + 1] = wgt_h
    o_ref[0, :, C + heads:] = jnp.zeros((R, 16 - heads), jnp.float32)


def _attention(gath, jp, sims, B, C, Kc, topk, heads, gk):
    ow = C + 16
    scale = (C // heads) ** -0.5
    R = gk * topk
    body = functools.partial(_attn_body, C=C, topk=topk, gk=gk, heads=heads,
                             scale=scale)
    grid = (B, Kc // gk)
    out = pl.pallas_call(
        body,
        grid=grid,
        in_specs=[
            pl.BlockSpec((1, R, 3 * C), lambda b, t: (b, t, 0)),
            pl.BlockSpec((1, gk, topk, topk), lambda b, t: (b, t, 0, 0)),
            pl.BlockSpec((1, gk, topk), lambda b, t: (b, t, 0)),
        ],
        out_specs=pl.BlockSpec((1, R, ow), lambda b, t: (b, t, 0)),
        out_shape=jax.ShapeDtypeStruct((B, Kc * topk, ow), jnp.float32),
        scratch_shapes=[pltpu.VMEM((R, R), jnp.float32),
                        pltpu.VMEM((1, R), jnp.float32)],
        interpret=_INTERPRET,
    )(gath.reshape(B, Kc * topk, 3 * C), jp, sims)
    return out


def _combine_body(nd_ref, vt_ref, res_ref, *, C, heads):
    hd = C // heads
    nd = nd_ref[0]
    num = nd[:, 0:C]
    vt = vt_ref[0]
    den = jnp.concatenate([
        jnp.broadcast_to(nd[:, C + h:C + h + 1], num[:, :hd].shape)
        for h in range(heads)], axis=1)
    res = jnp.where(den > 1e-6, num / jnp.maximum(den, 1e-6), vt)
    res_ref[0] = res.T


def _combine(numden, vt, B, C, N, heads, Nt):
    body = functools.partial(_combine_body, C=C, heads=heads)
    grid = (B, N // Nt)
    return pl.pallas_call(
        body,
        grid=grid,
        in_specs=[
            pl.BlockSpec((1, Nt, C + 16), lambda b, t: (b, t, 0)),
            pl.BlockSpec((1, Nt, C), lambda b, t: (b, t, 0)),
        ],
        out_specs=pl.BlockSpec((1, C, Nt), lambda b, t: (b, 0, t)),
        out_shape=jax.ShapeDtypeStruct((B, C, N), jnp.float32),
        interpret=_INTERPRET,
    )(numden, vt)


def _sc_gather(table, gidx, D):
    M = gidx.shape[0]
    per_w = M // _SC_WORKERS
    nch = per_w // _SC_CHUNK
    mesh = plsc.VectorSubcoreMesh(core_axis_name="c", subcore_axis_name="s")

    @functools.partial(
        pl.kernel, mesh=mesh,
        out_type=jax.ShapeDtypeStruct((M, D), jnp.float32),
        scratch_types=[pltpu.VMEM((_SC_CHUNK,), jnp.int32),
                       pltpu.VMEM((_SC_CHUNK, D), jnp.float32),
                       pltpu.SemaphoreType.DMA],
        compiler_params=pltpu.CompilerParams(use_tc_tiling_on_sc=False),
    )
    def k(table_hbm, gidx_hbm, out_hbm, idx_v, rows_v, sem):
        wid = lax.axis_index("s") * 2 + lax.axis_index("c")
        base = wid * per_w

        def chunk(i, carry):
            off = base + i * _SC_CHUNK
            pltpu.sync_copy(gidx_hbm.at[pl.ds(off, _SC_CHUNK)], idx_v)
            pltpu.async_copy(table_hbm.at[idx_v], rows_v, sem).wait()
            pltpu.sync_copy(rows_v, out_hbm.at[pl.ds(off, _SC_CHUNK)])
            return carry

        lax.fori_loop(0, nch, chunk, 0)

    return k(table, gidx)


def _sc_scatter(contrib, inds, zrow, N):
    B, R, OW = contrib.shape
    per_t = R // 16
    nch = per_t // _SC_CHUNK
    half = N // 2
    acc_per_t = half // 16
    nz = acc_per_t // _SC_CHUNK
    mesh = plsc.VectorSubcoreMesh(core_axis_name="c", subcore_axis_name="s")

    @functools.partial(
        pl.kernel, mesh=mesh,
        out_type=jax.ShapeDtypeStruct((B, N, OW), jnp.float32),
        scratch_types=[pltpu.VMEM((_SC_CHUNK,), jnp.int32),
                       pltpu.VMEM((_SC_CHUNK,), jnp.int32),
                       pltpu.VMEM((_SC_CHUNK, OW), jnp.float32),
                       pltpu.VMEM((_SC_CHUNK, OW), jnp.float32),
                       pltpu.VMEM_SHARED((half + _SC_CHUNK, OW), jnp.float32),
                       pltpu.SemaphoreType.DMA],
        compiler_params=pltpu.CompilerParams(use_tc_tiling_on_sc=False),
    )
    def k(contrib_hbm, inds_hbm, zrow_hbm, out_hbm, idx_v, idx2_v, rows_v,
          zbuf, acc, sem):
        cid = lax.axis_index("c")
        sid = lax.axis_index("s")
        pltpu.sync_copy(zrow_hbm, zbuf)
        for j in range(B // 2):
            b = cid * (B // 2) + j
            for lo in (0, half):

                def zero(i, carry):
                    pltpu.sync_copy(
                        zbuf, acc.at[pl.ds(sid * acc_per_t + i * _SC_CHUNK,
                                           _SC_CHUNK)])
                    return carry

                lax.fori_loop(0, nz, zero, 0)
                plsc.subcore_barrier()

                def chunk(i, carry):
                    off = sid * per_t + i * _SC_CHUNK
                    pltpu.sync_copy(inds_hbm.at[b].at[pl.ds(off, _SC_CHUNK)],
                                    idx_v)
                    pltpu.sync_copy(contrib_hbm.at[b].at[pl.ds(off,
                                                               _SC_CHUNK)],
                                    rows_v)
                    for t in range(_SC_CHUNK // 16):
                        iv = idx_v[pl.ds(t * 16, 16)]
                        inr = jnp.logical_and(iv >= lo, iv < lo + half)
                        adj = jnp.where(inr, iv - lo, half + (iv & 127))
                        idx2_v[pl.ds(t * 16, 16)] = adj
                    pltpu.sync_copy(rows_v, acc.at[idx2_v], add=True)
                    return carry

                lax.fori_loop(0, nch, chunk, 0)
                plsc.subcore_barrier()
                pltpu.sync_copy(
                    acc.at[pl.ds(sid * acc_per_t, acc_per_t)],
                    out_hbm.at[b].at[pl.ds(lo + sid * acc_per_t, acc_per_t)])
                plsc.subcore_barrier()

    return k(contrib, inds, zrow)


def kernel(x, joint_probs, norm_gamma, norm_beta, Wq, Wk, Wv):
    B, C, H, W = x.shape
    N = H * W
    Kc = joint_probs.shape[1]
    topk = joint_probs.shape[2]
    heads = 4
    gh = gw = int(np.sqrt(Kc))
    ph, pw = H // gh, W // gw

    Nt = 2048
    rows_per_tile = Nt // W
    cells_per_tile = (rows_per_tile // ph) * gw

    nloc = np.arange(Nt)
    hl, wl = nloc // W, nloc % W
    cell = (hl // ph) * gw + (wl // pw)
    pool_t = np.zeros((cells_per_tile, Nt), np.float32)
    pool_t[cell, nloc] = 1.0 / (ph * pw)
    pool_t = jnp.asarray(pool_t)

    xf = x.reshape(B, C, N)
    xn, qkvt, centt = _prep(
        xf, norm_gamma.reshape(C, 1), norm_beta.reshape(C, 1),
        Wq.T, Wk.T, Wv.T, pool_t, B, C, N, Nt, Kc)

    sims, inds = _topk(centt, xn, B, C, Kc, N, topk, rows=128)

    gidx = (inds.reshape(B, Kc * topk)
            + (jnp.arange(B, dtype=jnp.int32) * N)[:, None]).reshape(-1)
    gath = _sc_gather(qkvt.reshape(B * N, 3 * C), gidx, 3 * C)

    o = _attention(gath.reshape(B, Kc * topk, 3 * C), joint_probs, sims,
                   B, C, Kc, topk, heads, gk=8)

    zrow = jnp.zeros((_SC_CHUNK, C + 16), jnp.float32)
    numden = _sc_scatter(o, inds.reshape(B, Kc * topk), zrow, N)

    res = _combine(numden, qkvt[..., 2 * C:3 * C], B, C, N, heads, Nt=2048)
    return res.reshape(B, C, H, W)


# speedup vs baseline: 1.1217x; 1.0386x over previous
import functools

import jax
import jax.numpy as jnp
import numpy as np
from jax import lax
from jax.experimental import pallas as pl
from jax.experimental.pallas import tpu as pltpu
from jax.experimental.pallas import tpu_sc as plsc

_INTERPRET = False

_SC_CHUNK = 128
_SC_WORKERS = 32

_NEG = -3.0e38


def _prep_body(x_ref, g_ref, b_ref, wqt_ref, wkt_ref, wvt_ref, poolt_ref,
               xn_ref, qkvt_ref, centt_ref):
    xb = x_ref[0]
    mu = jnp.mean(xb, axis=0, keepdims=True)
    var = jnp.mean((xb - mu) * (xb - mu), axis=0, keepdims=True)
    xn = (xb - mu) / jnp.sqrt(var + 1e-6) * g_ref[:] + b_ref[:]
    xn_ref[0] = xn
    xnt = xn.T
    qt = jnp.dot(xnt, wqt_ref[:], preferred_element_type=jnp.float32)
    kt = jnp.dot(xnt, wkt_ref[:], preferred_element_type=jnp.float32)
    vt = jnp.dot(xnt, wvt_ref[:], preferred_element_type=jnp.float32)
    qkvt_ref[0] = jnp.concatenate([qt, kt, vt], axis=1)
    centt_ref[0] = jnp.dot(poolt_ref[:], xnt, preferred_element_type=jnp.float32,
                           precision=lax.Precision.HIGHEST)


def _prep(x, gamma, beta, Wq, Wk, Wv, pool_t, B, C, N, Nt, Kc):
    grid = (B, N // Nt)
    kcells = pool_t.shape[0]
    return pl.pallas_call(
        _prep_body,
        grid=grid,
        in_specs=[
            pl.BlockSpec((1, C, Nt), lambda b, t: (b, 0, t)),
            pl.BlockSpec((C, 1), lambda b, t: (0, 0)),
            pl.BlockSpec((C, 1), lambda b, t: (0, 0)),
            pl.BlockSpec((C, C), lambda b, t: (0, 0)),
            pl.BlockSpec((C, C), lambda b, t: (0, 0)),
            pl.BlockSpec((C, C), lambda b, t: (0, 0)),
            pl.BlockSpec((kcells, Nt), lambda b, t: (0, 0)),
        ],
        out_specs=[
            pl.BlockSpec((1, C, Nt), lambda b, t: (b, 0, t)),
            pl.BlockSpec((1, Nt, 3 * C), lambda b, t: (b, t, 0)),
            pl.BlockSpec((1, kcells, C), lambda b, t: (b, t, 0)),
        ],
        out_shape=[
            jax.ShapeDtypeStruct((B, C, N), jnp.float32),
            jax.ShapeDtypeStruct((B, N, 3 * C), jnp.float32),
            jax.ShapeDtypeStruct((B, Kc, C), jnp.float32),
        ],
        interpret=_INTERPRET,
    )(x, gamma, beta, Wq, Wk, Wv, pool_t)


def _topk_body(centt_ref, xn_ref, sims_ref, inds_ref, w_ref, *, n, topk, rows):
    w_ref[:] = jnp.dot(centt_ref[0], xn_ref[0],
                       preferred_element_type=jnp.float32)

    iota = lax.broadcasted_iota(jnp.int32, (rows, n), 1)
    for i in range(topk):
        w = w_ref[:]
        m = jnp.max(w, axis=1, keepdims=True)
        idx = jnp.min(jnp.where(w == m, iota, n), axis=1, keepdims=True)
        w_ref[:] = jnp.where(iota == idx, _NEG, w)
        sims_ref[0, :, i:i + 1] = m
        inds_ref[0, :, i:i + 1] = idx


def _topk(centt, xn, B, C, Kc, N, topk, rows):
    grid = (B * (Kc // rows),)
    nb = Kc // rows
    body = functools.partial(_topk_body, n=N, topk=topk, rows=rows)
    return pl.pallas_call(
        body,
        grid=grid,
        in_specs=[
            pl.BlockSpec((1, rows, C), lambda i: (i // nb, i % nb, 0)),
            pl.BlockSpec((1, C, N), lambda i: (i // nb, 0, 0)),
        ],
        out_specs=[
            pl.BlockSpec((1, rows, topk), lambda i: (i // nb, i % nb, 0)),
            pl.BlockSpec((1, rows, topk), lambda i: (i // nb, i % nb, 0)),
        ],
        out_shape=[
            jax.ShapeDtypeStruct((B, Kc, topk), jnp.float32),
            jax.ShapeDtypeStruct((B, Kc, topk), jnp.int32),
        ],
        scratch_shapes=[pltpu.VMEM((rows, N), jnp.float32)],
        interpret=_INTERPRET,
    )(centt, xn)


def _attn_body(g_ref, jp_ref, sims_ref, o_ref, jpd_ref, sflat_ref, *, C, topk,
               gk, heads, scale):
    R = gk * topk
    hd = C // heads
    g = g_ref[0]
    sims = sims_ref[0]
    sm = jnp.max(sims, axis=1, keepdims=True)
    se = jnp.exp(sims - sm)
    sims_n = se / jnp.sum(se, axis=1, keepdims=True)
    for s in range(gk):
        sflat_ref[0:1, s * topk:(s + 1) * topk] = sims_n[s:s + 1, :]
    sims_flat = sflat_ref[:]

    jpd_ref[:] = jnp.zeros((R, R), jnp.float32)
    for s in range(gk):
        jpd_ref[s * topk:(s + 1) * topk,
                s * topk:(s + 1) * topk] = jp_ref[0, s] * scale
    jpd = jpd_ref[:]

    bi = lax.broadcasted_iota(jnp.int32, (R, R), 0) // topk
    bj = lax.broadcasted_iota(jnp.int32, (R, R), 1) // topk
    mask = bi == bj

    q_all = g[:, 0:C]
    k_all = g[:, C:2 * C]
    v_all = g[:, 2 * C:3 * C]

    for h in range(heads):
        qh = q_all[:, h * hd:(h + 1) * hd]
        kh = k_all[:, h * hd:(h + 1) * hd]
        vh = v_all[:, h * hd:(h + 1) * hd]
        s_mat = lax.dot_general(qh, kh, (((1,), (1,)), ((), ())),
                                preferred_element_type=jnp.float32)
        s_mat = s_mat * jpd
        s_mat = jnp.where(mask, s_mat, _NEG)
        p = jnp.exp(s_mat - jnp.max(s_mat, axis=1, keepdims=True))
        a = p / jnp.sum(p, axis=1, keepdims=True)
        a = a * sims_flat
        out_h = jnp.dot(a, vh, preferred_element_type=jnp.float32)
        wgt_h = jnp.sum(a, axis=1, keepdims=True)
        o_ref[0, :, h * hd:(h + 1) * hd] = out_h
        o_ref[0, :, C + h:C + h + 1] = wgt_h
    o_ref[0, :, C + heads:] = jnp.zeros((R, 16 - heads), jnp.float32)


def _attention(gath, jp, sims, B, C, Kc, topk, heads, gk):
    ow = C + 16
    scale = (C // heads) ** -0.5
    R = gk * topk
    body = functools.partial(_attn_body, C=C, topk=topk, gk=gk, heads=heads,
                             scale=scale)
    grid = (B, Kc // gk)
    out = pl.pallas_call(
        body,
        grid=grid,
        in_specs=[
            pl.BlockSpec((1, R, 3 * C), lambda b, t: (b, t, 0)),
            pl.BlockSpec((1, gk, topk, topk), lambda b, t: (b, t, 0, 0)),
            pl.BlockSpec((1, gk, topk), lambda b, t: (b, t, 0)),
        ],
        out_specs=pl.BlockSpec((1, R, ow), lambda b, t: (b, t, 0)),
        out_shape=jax.ShapeDtypeStruct((B, Kc * topk, ow), jnp.float32),
        scratch_shapes=[pltpu.VMEM((R, R), jnp.float32),
                        pltpu.VMEM((1, R), jnp.float32)],
        interpret=_INTERPRET,
    )(gath.reshape(B, Kc * topk, 3 * C), jp, sims)
    return out


def _combine_body(nd_ref, vt_ref, res_ref, *, C, heads):
    hd = C // heads
    nd = nd_ref[0]
    num = nd[:, 0:C]
    vt = vt_ref[0]
    den = jnp.concatenate([
        jnp.broadcast_to(nd[:, C + h:C + h + 1], num[:, :hd].shape)
        for h in range(heads)], axis=1)
    res = jnp.where(den > 1e-6, num / jnp.maximum(den, 1e-6), vt)
    res_ref[0] = res.T


def _combine(numden, vt, B, C, N, heads, Nt):
    body = functools.partial(_combine_body, C=C, heads=heads)
    grid = (B, N // Nt)
    return pl.pallas_call(
        body,
        grid=grid,
        in_specs=[
            pl.BlockSpec((1, Nt, C + 16), lambda b, t: (b, t, 0)),
            pl.BlockSpec((1, Nt, C), lambda b, t: (b, t, 0)),
        ],
        out_specs=pl.BlockSpec((1, C, Nt), lambda b, t: (b, 0, t)),
        out_shape=jax.ShapeDtypeStruct((B, C, N), jnp.float32),
        interpret=_INTERPRET,
    )(numden, vt)


def _sc_gather(table, gidx, D):
    M = gidx.shape[0]
    per_w = M // _SC_WORKERS
    nch = per_w // _SC_CHUNK
    mesh = plsc.VectorSubcoreMesh(core_axis_name="c", subcore_axis_name="s")

    @functools.partial(
        pl.kernel, mesh=mesh,
        out_type=jax.ShapeDtypeStruct((M, D), jnp.float32),
        scratch_types=[pltpu.VMEM((_SC_CHUNK,), jnp.int32),
                       pltpu.VMEM((_SC_CHUNK, D), jnp.float32),
                       pltpu.SemaphoreType.DMA],
        compiler_params=pltpu.CompilerParams(use_tc_tiling_on_sc=False),
    )
    def k(table_hbm, gidx_hbm, out_hbm, idx_v, rows_v, sem):
        wid = lax.axis_index("s") * 2 + lax.axis_index("c")
        base = wid * per_w

        def chunk(i, carry):
            off = base + i * _SC_CHUNK
            pltpu.sync_copy(gidx_hbm.at[pl.ds(off, _SC_CHUNK)], idx_v)
            pltpu.async_copy(table_hbm.at[idx_v], rows_v, sem).wait()
            pltpu.sync_copy(rows_v, out_hbm.at[pl.ds(off, _SC_CHUNK)])
            return carry

        lax.fori_loop(0, nch, chunk, 0)

    return k(table, gidx)


def _sc_scatter(contrib, inds, zrow, N):
    B, R, OW = contrib.shape
    per_t = R // 16
    nch = per_t // _SC_CHUNK
    half = N // 2
    acc_per_t = half // 16
    nz = acc_per_t // _SC_CHUNK
    mesh = plsc.VectorSubcoreMesh(core_axis_name="c", subcore_axis_name="s")

    @functools.partial(
        pl.kernel, mesh=mesh,
        out_type=jax.ShapeDtypeStruct((B, N, OW), jnp.float32),
        scratch_types=[pltpu.VMEM((_SC_CHUNK,), jnp.int32),
                       pltpu.VMEM((_SC_CHUNK,), jnp.int32),
                       pltpu.VMEM((_SC_CHUNK, OW), jnp.float32),
                       pltpu.VMEM((_SC_CHUNK, OW), jnp.float32),
                       pltpu.VMEM_SHARED((half + _SC_CHUNK, OW), jnp.float32),
                       pltpu.SemaphoreType.DMA],
        compiler_params=pltpu.CompilerParams(use_tc_tiling_on_sc=False),
    )
    def k(contrib_hbm, inds_hbm, zrow_hbm, out_hbm, idx_v, idx2_v, rows_v,
          zbuf, acc, sem):
        cid = lax.axis_index("c")
        sid = lax.axis_index("s")
        pltpu.sync_copy(zrow_hbm, zbuf)
        for j in range(B // 2):
            b = cid * (B // 2) + j
            for lo in (0, half):

                def zero(i, carry):
                    pltpu.sync_copy(
                        zbuf, acc.at[pl.ds(sid * acc_per_t + i * _SC_CHUNK,
                                           _SC_CHUNK)])
                    return carry

                lax.fori_loop(0, nz, zero, 0)
                plsc.subcore_barrier()

                def chunk(i, carry):
                    off = sid * per_t + i * _SC_CHUNK
                    pltpu.sync_copy(inds_hbm.at[b].at[pl.ds(off, _SC_CHUNK)],
                                    idx_v)
                    pltpu.sync_copy(contrib_hbm.at[b].at[pl.ds(off,
                                                               _SC_CHUNK)],
                                    rows_v)
                    for t in range(_SC_CHUNK // 16):
                        iv = idx_v[pl.ds(t * 16, 16)]
                        inr = jnp.logical_and(iv >= lo, iv < lo + half)
                        adj = jnp.where(inr, iv - lo, half + (iv & 127))
                        idx2_v[pl.ds(t * 16, 16)] = adj
                    pltpu.sync_copy(rows_v, acc.at[idx2_v], add=True)
                    return carry

                lax.fori_loop(0, nch, chunk, 0)
                plsc.subcore_barrier()
                pltpu.sync_copy(
                    acc.at[pl.ds(sid * acc_per_t, acc_per_t)],
                    out_hbm.at[b].at[pl.ds(lo + sid * acc_per_t, acc_per_t)])
                plsc.subcore_barrier()

    return k(contrib, inds, zrow)


def kernel(x, joint_probs, norm_gamma, norm_beta, Wq, Wk, Wv):
    B, C, H, W = x.shape
    N = H * W
    Kc = joint_probs.shape[1]
    topk = joint_probs.shape[2]
    heads = 4
    gh = gw = int(np.sqrt(Kc))
    ph, pw = H // gh, W // gw

    Nt = 2048
    rows_per_tile = Nt // W
    cells_per_tile = (rows_per_tile // ph) * gw

    nloc = np.arange(Nt)
    hl, wl = nloc // W, nloc % W
    cell = (hl // ph) * gw + (wl // pw)
    pool_t = np.zeros((cells_per_tile, Nt), np.float32)
    pool_t[cell, nloc] = 1.0 / (ph * pw)
    pool_t = jnp.asarray(pool_t)

    xf = x.reshape(B, C, N)
    xn, qkvt, centt = _prep(
        xf, norm_gamma.reshape(C, 1), norm_beta.reshape(C, 1),
        Wq.T, Wk.T, Wv.T, pool_t, B, C, N, Nt, Kc)

    sims, inds = _topk(centt, xn, B, C, Kc, N, topk, rows=256)

    gidx = (inds.reshape(B, Kc * topk)
            + (jnp.arange(B, dtype=jnp.int32) * N)[:, None]).reshape(-1)
    gath = _sc_gather(qkvt.reshape(B * N, 3 * C), gidx, 3 * C)

    o = _attention(gath.reshape(B, Kc * topk, 3 * C), joint_probs, sims,
                   B, C, Kc, topk, heads, gk=8)

    zrow = jnp.zeros((_SC_CHUNK, C + 16), jnp.float32)
    numden = _sc_scatter(o, inds.reshape(B, Kc * topk), zrow, N)

    res = _combine(numden, qkvt[..., 2 * C:3 * C], B, C, N, heads, Nt=2048)
    return res.reshape(B, C, H, W)


# speedup vs baseline: 1.1352x; 1.0120x over previous
import functools

import jax
import jax.numpy as jnp
import numpy as np
from jax import lax
from jax.experimental import pallas as pl
from jax.experimental.pallas import tpu as pltpu
from jax.experimental.pallas import tpu_sc as plsc

_INTERPRET = False

_SC_CHUNK = 128
_SC_WORKERS = 32

_NEG = -3.0e38


def _prep_body(x_ref, g_ref, b_ref, wqt_ref, wkt_ref, wvt_ref, poolt_ref,
               xn_ref, qkvt_ref, centt_ref):
    xb = x_ref[0]
    mu = jnp.mean(xb, axis=0, keepdims=True)
    var = jnp.mean((xb - mu) * (xb - mu), axis=0, keepdims=True)
    xn = (xb - mu) / jnp.sqrt(var + 1e-6) * g_ref[:] + b_ref[:]
    xn_ref[0] = xn
    xnt = xn.T
    qt = jnp.dot(xnt, wqt_ref[:], preferred_element_type=jnp.float32)
    kt = jnp.dot(xnt, wkt_ref[:], preferred_element_type=jnp.float32)
    vt = jnp.dot(xnt, wvt_ref[:], preferred_element_type=jnp.float32)
    qkvt_ref[0] = jnp.concatenate([qt, kt, vt], axis=1)
    centt_ref[0] = jnp.dot(poolt_ref[:], xnt, preferred_element_type=jnp.float32,
                           precision=lax.Precision.HIGHEST)


def _prep(x, gamma, beta, Wq, Wk, Wv, pool_t, B, C, N, Nt, Kc):
    grid = (B, N // Nt)
    kcells = pool_t.shape[0]
    return pl.pallas_call(
        _prep_body,
        grid=grid,
        in_specs=[
            pl.BlockSpec((1, C, Nt), lambda b, t: (b, 0, t)),
            pl.BlockSpec((C, 1), lambda b, t: (0, 0)),
            pl.BlockSpec((C, 1), lambda b, t: (0, 0)),
            pl.BlockSpec((C, C), lambda b, t: (0, 0)),
            pl.BlockSpec((C, C), lambda b, t: (0, 0)),
            pl.BlockSpec((C, C), lambda b, t: (0, 0)),
            pl.BlockSpec((kcells, Nt), lambda b, t: (0, 0)),
        ],
        out_specs=[
            pl.BlockSpec((1, C, Nt), lambda b, t: (b, 0, t)),
            pl.BlockSpec((1, Nt, 3 * C), lambda b, t: (b, t, 0)),
            pl.BlockSpec((1, kcells, C), lambda b, t: (b, t, 0)),
        ],
        out_shape=[
            jax.ShapeDtypeStruct((B, C, N), jnp.float32),
            jax.ShapeDtypeStruct((B, N, 3 * C), jnp.float32),
            jax.ShapeDtypeStruct((B, Kc, C), jnp.float32),
        ],
        interpret=_INTERPRET,
    )(x, gamma, beta, Wq, Wk, Wv, pool_t)


def _topk_body(centt_ref, xn_ref, sims_ref, inds_ref, w_ref, *, n, topk, rows):
    w_ref[:] = jnp.dot(centt_ref[0], xn_ref[0],
                       preferred_element_type=jnp.float32)

    iota = lax.broadcasted_iota(jnp.int32, (rows, n), 1)
    for i in range(topk):
        w = w_ref[:]
        m = jnp.max(w, axis=1, keepdims=True)
        idx = jnp.min(jnp.where(w == m, iota, n), axis=1, keepdims=True)
        w_ref[:] = jnp.where(iota == idx, _NEG, w)
        sims_ref[0, :, i:i + 1] = m
        inds_ref[0, :, i:i + 1] = idx


def _topk(centt, xn, B, C, Kc, N, topk, rows):
    grid = (B * (Kc // rows),)
    nb = Kc // rows
    body = functools.partial(_topk_body, n=N, topk=topk, rows=rows)
    return pl.pallas_call(
        body,
        grid=grid,
        in_specs=[
            pl.BlockSpec((1, rows, C), lambda i: (i // nb, i % nb, 0)),
            pl.BlockSpec((1, C, N), lambda i: (i // nb, 0, 0)),
        ],
        out_specs=[
            pl.BlockSpec((1, rows, topk), lambda i: (i // nb, i % nb, 0)),
            pl.BlockSpec((1, rows, topk), lambda i: (i // nb, i % nb, 0)),
        ],
        out_shape=[
            jax.ShapeDtypeStruct((B, Kc, topk), jnp.float32),
            jax.ShapeDtypeStruct((B, Kc, topk), jnp.int32),
        ],
        scratch_shapes=[pltpu.VMEM((rows, N), jnp.float32)],
        interpret=_INTERPRET,
    )(centt, xn)


def _attn_body(g_ref, jp_ref, sims_ref, o_ref, jpd_ref, sflat_ref, *, C, topk,
               gk, heads, scale):
    R = gk * topk
    hd = C // heads
    g = g_ref[0]
    sims = sims_ref[0]
    sm = jnp.max(sims, axis=1, keepdims=True)
    se = jnp.exp(sims - sm)
    sims_n = se / jnp.sum(se, axis=1, keepdims=True)
    for s in range(gk):
        sflat_ref[0:1, s * topk:(s + 1) * topk] = sims_n[s:s + 1, :]
    sims_flat = sflat_ref[:]

    jpd_ref[:] = jnp.zeros((R, R), jnp.float32)
    for s in range(gk):
        jpd_ref[s * topk:(s + 1) * topk,
                s * topk:(s + 1) * topk] = jp_ref[0, s] * scale
    jpd = jpd_ref[:]

    bi = lax.broadcasted_iota(jnp.int32, (R, R), 0) // topk
    bj = lax.broadcasted_iota(jnp.int32, (R, R), 1) // topk
    mask = bi == bj

    q_all = g[:, 0:C]
    k_all = g[:, C:2 * C]
    v_all = g[:, 2 * C:3 * C]

    for h in range(heads):
        qh = q_all[:, h * hd:(h + 1) * hd]
        kh = k_all[:, h * hd:(h + 1) * hd]
        vh = v_all[:, h * hd:(h + 1) * hd]
        s_mat = lax.dot_general(qh, kh, (((1,), (1,)), ((), ())),
                                preferred_element_type=jnp.float32)
        s_mat = s_mat * jpd
        s_mat = jnp.where(mask, s_mat, _NEG)
        p = jnp.exp(s_mat - jnp.max(s_mat, axis=1, keepdims=True))
        a = p / jnp.sum(p, axis=1, keepdims=True)
        a = a * sims_flat
        out_h = jnp.dot(a, vh, preferred_element_type=jnp.float32)
        wgt_h = jnp.sum(a, axis=1, keepdims=True)
        o_ref[0, :, h * hd:(h + 1) * hd] = out_h
        o_ref[0, :, C + h:C + h + 1] = wgt_h
    o_ref[0, :, C + heads:] = jnp.zeros((R, 16 - heads), jnp.float32)


def _attention(gath, jp, sims, B, C, Kc, topk, heads, gk):
    ow = C + 16
    scale = (C // heads) ** -0.5
    R = gk * topk
    body = functools.partial(_attn_body, C=C, topk=topk, gk=gk, heads=heads,
                             scale=scale)
    grid = (B, Kc // gk)
    out = pl.pallas_call(
        body,
        grid=grid,
        in_specs=[
            pl.BlockSpec((1, R, 3 * C), lambda b, t: (b, t, 0)),
            pl.BlockSpec((1, gk, topk, topk), lambda b, t: (b, t, 0, 0)),
            pl.BlockSpec((1, gk, topk), lambda b, t: (b, t, 0)),
        ],
        out_specs=pl.BlockSpec((1, R, ow), lambda b, t: (b, t, 0)),
        out_shape=jax.ShapeDtypeStruct((B, Kc * topk, ow), jnp.float32),
        scratch_shapes=[pltpu.VMEM((R, R), jnp.float32),
                        pltpu.VMEM((1, R), jnp.float32)],
        interpret=_INTERPRET,
    )(gath.reshape(B, Kc * topk, 3 * C), jp, sims)
    return out


def _combine_body(nd_ref, vt_ref, res_ref, *, C, heads):
    hd = C // heads
    nd = nd_ref[0]
    num = nd[:, 0:C]
    vt = vt_ref[0]
    den = jnp.concatenate([
        jnp.broadcast_to(nd[:, C + h:C + h + 1], num[:, :hd].shape)
        for h in range(heads)], axis=1)
    res = jnp.where(den > 1e-6, num / jnp.maximum(den, 1e-6), vt)
    res_ref[0] = res.T


def _combine(numden, vt, B, C, N, heads, Nt):
    body = functools.partial(_combine_body, C=C, heads=heads)
    grid = (B, N // Nt)
    return pl.pallas_call(
        body,
        grid=grid,
        in_specs=[
            pl.BlockSpec((1, Nt, C + 16), lambda b, t: (b, t, 0)),
            pl.BlockSpec((1, Nt, C), lambda b, t: (b, t, 0)),
        ],
        out_specs=pl.BlockSpec((1, C, Nt), lambda b, t: (b, 0, t)),
        out_shape=jax.ShapeDtypeStruct((B, C, N), jnp.float32),
        interpret=_INTERPRET,
    )(numden, vt)


def _sc_gather(table, gidx, D):
    M = gidx.shape[0]
    per_w = M // _SC_WORKERS
    nch = per_w // _SC_CHUNK
    mesh = plsc.VectorSubcoreMesh(core_axis_name="c", subcore_axis_name="s")

    @functools.partial(
        pl.kernel, mesh=mesh,
        out_type=jax.ShapeDtypeStruct((M, D), jnp.float32),
        scratch_types=[pltpu.VMEM((_SC_CHUNK,), jnp.int32),
                       pltpu.VMEM((_SC_CHUNK, D), jnp.float32),
                       pltpu.SemaphoreType.DMA],
        compiler_params=pltpu.CompilerParams(use_tc_tiling_on_sc=False),
    )
    def k(table_hbm, gidx_hbm, out_hbm, idx_v, rows_v, sem):
        wid = lax.axis_index("s") * 2 + lax.axis_index("c")
        base = wid * per_w

        def chunk(i, carry):
            off = base + i * _SC_CHUNK
            pltpu.sync_copy(gidx_hbm.at[pl.ds(off, _SC_CHUNK)], idx_v)
            pltpu.async_copy(table_hbm.at[idx_v], rows_v, sem).wait()
            pltpu.sync_copy(rows_v, out_hbm.at[pl.ds(off, _SC_CHUNK)])
            return carry

        lax.fori_loop(0, nch, chunk, 0)

    return k(table, gidx)


def _sc_scatter(contrib, inds, zrow, N):
    B, R, OW = contrib.shape
    per_t = R // 16
    nch = per_t // _SC_CHUNK
    half = N // 2
    acc_per_t = half // 16
    nz = acc_per_t // _SC_CHUNK
    mesh = plsc.VectorSubcoreMesh(core_axis_name="c", subcore_axis_name="s")

    @functools.partial(
        pl.kernel, mesh=mesh,
        out_type=jax.ShapeDtypeStruct((B, N, OW), jnp.float32),
        scratch_types=[pltpu.VMEM((_SC_CHUNK,), jnp.int32),
                       pltpu.VMEM((_SC_CHUNK,), jnp.int32),
                       pltpu.VMEM((_SC_CHUNK, OW), jnp.float32),
                       pltpu.VMEM((_SC_CHUNK, OW), jnp.float32),
                       pltpu.VMEM_SHARED((half + _SC_CHUNK, OW), jnp.float32),
                       pltpu.SemaphoreType.DMA],
        compiler_params=pltpu.CompilerParams(use_tc_tiling_on_sc=False),
    )
    def k(contrib_hbm, inds_hbm, zrow_hbm, out_hbm, idx_v, idx2_v, rows_v,
          zbuf, acc, sem):
        cid = lax.axis_index("c")
        sid = lax.axis_index("s")
        pltpu.sync_copy(zrow_hbm, zbuf)
        for j in range(B // 2):
            b = cid * (B // 2) + j
            for lo in (0, half):

                def zero(i, carry):
                    pltpu.sync_copy(
                        zbuf, acc.at[pl.ds(sid * acc_per_t + i * _SC_CHUNK,
                                           _SC_CHUNK)])
                    return carry

                lax.fori_loop(0, nz, zero, 0)
                plsc.subcore_barrier()

                def chunk(i, carry):
                    off = sid * per_t + i * _SC_CHUNK
                    pltpu.sync_copy(inds_hbm.at[b].at[pl.ds(off, _SC_CHUNK)],
                                    idx_v)
                    pltpu.sync_copy(contrib_hbm.at[b].at[pl.ds(off,
                                                               _SC_CHUNK)],
                                    rows_v)
                    for t in range(_SC_CHUNK // 16):
                        iv = idx_v[pl.ds(t * 16, 16)]
                        inr = jnp.logical_and(iv >= lo, iv < lo + half)
                        adj = jnp.where(inr, iv - lo, half + (iv & 127))
                        idx2_v[pl.ds(t * 16, 16)] = adj
                    pltpu.sync_copy(rows_v, acc.at[idx2_v], add=True)
                    return carry

                lax.fori_loop(0, nch, chunk, 0)
                plsc.subcore_barrier()
                pltpu.sync_copy(
                    acc.at[pl.ds(sid * acc_per_t, acc_per_t)],
                    out_hbm.at[b].at[pl.ds(lo + sid * acc_per_t, acc_per_t)])
                plsc.subcore_barrier()

    return k(contrib, inds, zrow)


def kernel(x, joint_probs, norm_gamma, norm_beta, Wq, Wk, Wv):
    B, C, H, W = x.shape
    N = H * W
    Kc = joint_probs.shape[1]
    topk = joint_probs.shape[2]
    heads = 4
    gh = gw = int(np.sqrt(Kc))
    ph, pw = H // gh, W // gw

    Nt = 4096
    rows_per_tile = Nt // W
    cells_per_tile = (rows_per_tile // ph) * gw

    nloc = np.arange(Nt)
    hl, wl = nloc // W, nloc % W
    cell = (hl // ph) * gw + (wl // pw)
    pool_t = np.zeros((cells_per_tile, Nt), np.float32)
    pool_t[cell, nloc] = 1.0 / (ph * pw)
    pool_t = jnp.asarray(pool_t)

    xf = x.reshape(B, C, N)
    xn, qkvt, centt = _prep(
        xf, norm_gamma.reshape(C, 1), norm_beta.reshape(C, 1),
        Wq.T, Wk.T, Wv.T, pool_t, B, C, N, Nt, Kc)

    sims, inds = _topk(centt, xn, B, C, Kc, N, topk, rows=256)

    gidx = (inds.reshape(B, Kc * topk)
            + (jnp.arange(B, dtype=jnp.int32) * N)[:, None]).reshape(-1)
    gath = _sc_gather(qkvt.reshape(B * N, 3 * C), gidx, 3 * C)

    o = _attention(gath.reshape(B, Kc * topk, 3 * C), joint_probs, sims,
                   B, C, Kc, topk, heads, gk=8)

    zrow = jnp.zeros((_SC_CHUNK, C + 16), jnp.float32)
    numden = _sc_scatter(o, inds.reshape(B, Kc * topk), zrow, N)

    res = _combine(numden, qkvt[..., 2 * C:3 * C], B, C, N, heads, Nt=4096)
    return res.reshape(B, C, H, W)
